# baseline (speedup 1.0000x reference)
import functools

import jax
import jax.numpy as jnp
from jax import lax
from jax.experimental import pallas as pl
from jax.experimental.pallas import tpu as pltpu
from jax.experimental.pallas import tpu_sc as plsc

N_NODES = 10000
N_EDGES = 320000
D = 128
DH = D // 2

NC = 2
NS = 16
E_PER_T = N_EDGES // NS
C = 128
N_FULL = E_PER_T // C
C_TAIL = E_PER_T - N_FULL * C
NBUF = 4
RCHUNK = 200
N_RCHUNK = N_NODES // RCHUNK


def _fin_body(p_ref, w_ref, b_ref, o_ref):
    h = jnp.dot(p_ref[0], w_ref[:DH, :], preferred_element_type=jnp.float32)
    h += jnp.dot(p_ref[1], w_ref[DH:, :], preferred_element_type=jnp.float32)
    o_ref[...] = jnp.maximum(h + b_ref[...], 0.0)


def _finish(halves, W, b2d):
    return pl.pallas_call(
        _fin_body,
        grid=(5,),
        in_specs=[
            pl.BlockSpec((NC, 2000, DH), lambda i: (0, i, 0)),
            pl.BlockSpec((D, D), lambda i: (0, 0)),
            pl.BlockSpec((1, D), lambda i: (0, 0)),
        ],
        out_specs=pl.BlockSpec((2000, D), lambda i: (i, 0)),
        out_shape=jax.ShapeDtypeStruct((N_NODES, D), jnp.float32),
    )(halves, W, b2d)


def _sc_aggregate(edges, xa, xb):
    mesh = plsc.VectorSubcoreMesh(core_axis_name="c", subcore_axis_name="s")

    @functools.partial(
        pl.kernel,
        out_type=jax.ShapeDtypeStruct((NC, N_NODES, DH), jnp.float32),
        mesh=mesh,
        compiler_params=pltpu.CompilerParams(use_tc_tiling_on_sc=False),
        scratch_types=[
            pltpu.VMEM((E_PER_T,), jnp.int32),
            pltpu.VMEM((E_PER_T,), jnp.int32),
            pltpu.VMEM((C,), jnp.int32),
            pltpu.VMEM((C_TAIL,), jnp.int32),
            pltpu.VMEM((NBUF, C, DH), jnp.float32),
            pltpu.VMEM((C_TAIL, DH), jnp.float32),
            pltpu.VMEM((RCHUNK, DH), jnp.float32),
            pltpu.VMEM_SHARED((N_NODES, DH), jnp.float32),
            [pltpu.SemaphoreType.DMA] * NBUF,
        ],
    )
    def body(edges_hbm, xa_hbm, xb_hbm, out_hbm, src_all, dst_all, dst_cur,
             dst_tail, msg, msg_tail, robuf, agg, sems):
        c = lax.axis_index("c")
        s = lax.axis_index("s")

        ebase = pl.multiple_of(s * E_PER_T, 8)
        pltpu.async_copy(edges_hbm.at[pl.ds(ebase, E_PER_T)], src_all, sems[0])
        pltpu.async_copy(edges_hbm.at[pl.ds(N_EDGES + ebase, E_PER_T)],
                         dst_all, sems[1])

        def zrow(r, carry):
            for k in range(DH // 16):
                robuf[r, pl.ds(k * 16, 16)] = jnp.zeros((16,), jnp.float32)
            return carry

        lax.fori_loop(0, RCHUNK, zrow, 0)
        for t in range((N_RCHUNK + NS - 1) // NS):
            j = s + t * NS

            @pl.when(j < N_RCHUNK)
            def _():
                r0 = pl.multiple_of(j * RCHUNK, 8)
                pltpu.sync_copy(robuf, agg.at[pl.ds(r0, RCHUNK)])

        pltpu.make_async_copy(edges_hbm.at[pl.ds(ebase, E_PER_T)],
                              src_all, sems[0]).wait()
        pltpu.make_async_copy(edges_hbm.at[pl.ds(ebase, E_PER_T)],
                              dst_all, sems[1]).wait()
        plsc.subcore_barrier()

        def gather(j, buf):
            off = pl.multiple_of(j * C, 8)
            idx = src_all.at[pl.ds(off, C)]

            @pl.when(c == 0)
            def _():
                pltpu.async_copy(xa_hbm.at[idx], msg.at[buf], sems[buf])

            @pl.when(c == 1)
            def _():
                pltpu.async_copy(xb_hbm.at[idx], msg.at[buf], sems[buf])

        def wait(j, buf):
            off = pl.multiple_of(j * C, 8)
            pltpu.make_async_copy(xa_hbm.at[src_all.at[pl.ds(off, C)]],
                                  msg.at[buf], sems[buf]).wait()

        def scatter(j, buf):
            off = pl.multiple_of(j * C, 8)
            for t in range(C // 16):
                dst_cur[pl.ds(t * 16, 16)] = dst_all[pl.ds(off + t * 16, 16)]
            pltpu.sync_copy(msg.at[buf], agg.at[dst_cur], add=True)

        for b in range(NBUF - 1):
            gather(b, b)

        def group(g, carry):
            j0 = NBUF * g
            for b in range(NBUF):
                jn = j0 + b + (NBUF - 1)

                @pl.when(jn < N_FULL)
                def _():
                    gather(jn, (b + NBUF - 1) % NBUF)

                wait(j0 + b, b)
                scatter(j0 + b, b)
            return carry

        lax.fori_loop(0, N_FULL // NBUF, group, 0)

        toff = pl.multiple_of(N_FULL * C, 8)
        tidx = src_all.at[pl.ds(toff, C_TAIL)]

        @pl.when(c == 0)
        def _():
            pltpu.async_copy(xa_hbm.at[tidx], msg_tail, sems[0])

        @pl.when(c == 1)
        def _():
            pltpu.async_copy(xb_hbm.at[tidx], msg_tail, sems[0])

        pltpu.make_async_copy(xa_hbm.at[tidx], msg_tail, sems[0]).wait()
        for t in range(C_TAIL // 16):
            dst_tail[pl.ds(t * 16, 16)] = dst_all[pl.ds(toff + t * 16, 16)]
        pltpu.sync_copy(msg_tail, agg.at[dst_tail], add=True)

        plsc.subcore_barrier()
        for t in range((N_RCHUNK + NS - 1) // NS):
            j = s + t * NS

            @pl.when(j < N_RCHUNK)
            def _():
                r0 = pl.multiple_of(j * RCHUNK, 8)
                pltpu.sync_copy(agg.at[pl.ds(r0, RCHUNK)], robuf)
                pltpu.sync_copy(robuf, out_hbm.at[c, pl.ds(r0, RCHUNK)])

    return body(edges, xa, xb)


def kernel(inputs, adjacencies, W, b):
    halves = _sc_aggregate(adjacencies.reshape(-1),
                           inputs[:, :DH], inputs[:, DH:])
    return _finish(halves, W, b.reshape(1, D))
